# Initial kernel scaffold; baseline (speedup 1.0000x reference)
import jax
import jax.numpy as jnp
from jax.experimental import pallas as pl


def kernel(dense_inputs, sparse_inputs, seq_inputs, item_inputs, embed_sparse, embed_seq, att_W1, att_b1, att_a1, att_W2, att_b2, att_a2, att_Wout, att_bout, bn_gamma, bn_beta, ffn_W1, ffn_b1, ffn_a1, ffn_W2, ffn_b2, ffn_a2, Wout, bout):
    raise NotImplementedError("write your pallas kernel here")


# speedup vs baseline: 1.7501x; 1.7501x over previous
import functools

import jax
import jax.numpy as jnp
from jax import lax
from jax.experimental import pallas as pl
from jax.experimental.pallas import tpu as pltpu
from jax.experimental.pallas import tpu_sc as plsc

B = 4096
L = 200
V = 100000
D = 16
NS = 25
ND = 13

_NC = 2
_NSC = 16
_NW = _NC * _NSC

_G = 128
_SEQ_ROWS = B * L // _G
_SEQ_PW = _SEQ_ROWS // _NW
_K = 8
_SEQ_STEPS = _SEQ_PW // _K
_SP_ROWS = B * NS // _G
_SP_PW = _SP_ROWS // _NW
_IT_ROWS = B // _G


def _sc_gather_body(seq_tbl, sp_tbl, seq_idx, item_idx, sp_idx,
                    seq_out, item_out, sp_out,
                    idx_s, rows_s, idx_p, rows_p, idx_i, rows_i, sem):
    wid = lax.axis_index("s") * _NC + lax.axis_index("c")

    seq_base = wid * _SEQ_PW

    def seq_step(i, carry):
        base = seq_base + i * _K
        pltpu.sync_copy(seq_idx.at[pl.ds(base, _K)], idx_s)
        cps = [pltpu.async_copy(seq_tbl.at[idx_s.at[j]], rows_s.at[j], sem)
               for j in range(_K)]
        for c in cps:
            c.wait()
        pltpu.sync_copy(rows_s, seq_out.at[pl.ds(base, _K)])
        return carry

    lax.fori_loop(0, _SEQ_STEPS, seq_step, 0)

    pltpu.sync_copy(sp_idx.at[wid], idx_p)
    cps = [pltpu.async_copy(sp_tbl.at[idx_p.at[j]], rows_p.at[j], sem)
           for j in range(13)]
    for c in cps:
        c.wait()
    cps = [pltpu.async_copy(sp_tbl.at[idx_p.at[j]], rows_p.at[j], sem)
           for j in range(13, _SP_PW)]
    for c in cps:
        c.wait()
    pltpu.sync_copy(rows_p, sp_out.at[pl.ds(wid * _SP_PW, _SP_PW)])

    pltpu.sync_copy(item_idx.at[wid], idx_i)
    pltpu.async_copy(seq_tbl.at[idx_i.at[0]], rows_i.at[0], sem).wait()
    pltpu.sync_copy(rows_i, item_out.at[pl.ds(wid, 1)])


@functools.cache
def _build_sc_gather():
    return pl.kernel(
        _sc_gather_body,
        out_type=(
            jax.ShapeDtypeStruct((_SEQ_ROWS, _G, D), jnp.float32),
            jax.ShapeDtypeStruct((_IT_ROWS, _G, D), jnp.float32),
            jax.ShapeDtypeStruct((_SP_ROWS, _G, D), jnp.float32),
        ),
        mesh=plsc.VectorSubcoreMesh(core_axis_name="c", subcore_axis_name="s",
                                    num_cores=_NC, num_subcores=_NSC),
        compiler_params=pltpu.CompilerParams(use_tc_tiling_on_sc=False),
        scratch_types=[
            pltpu.VMEM((_K, _G), jnp.int32),
            pltpu.VMEM((_K, _G, D), jnp.float32),
            pltpu.VMEM((_SP_PW, _G), jnp.int32),
            pltpu.VMEM((_SP_PW, _G, D), jnp.float32),
            pltpu.VMEM((1, _G), jnp.int32),
            pltpu.VMEM((1, _G, D), jnp.float32),
            pltpu.SemaphoreType.DMA,
        ],
    )


_TB = 32
_M = _TB * L


def _prelu(x, a):
    return jnp.where(x > 0, x, a * x)


def _tc_body(seq_ref, ids_ref, it_ref, dn_ref, sp_ref,
             wq_ref, wk_ref, wqk_ref, b1_ref, a1_ref,
             w2_ref, b2_ref, a2_ref, wo_ref, bo_ref,
             fwu_ref, fwi_ref, fwd_ref, fws_ref, fb1_ref, fa1_ref,
             fw2_ref, fb2_ref, fa2_ref, fwo_ref, fbo_ref,
             out_ref):
    f32 = jnp.float32
    it = it_ref[...]
    k = seq_ref[...]
    A = jnp.dot(it, wq_ref[...], preferred_element_type=f32)
    q = jnp.broadcast_to(it[:, None, :], (_TB, L, D)).reshape(_M, D)
    Ab = jnp.broadcast_to(A[:, None, :], (_TB, L, 80)).reshape(_M, 80)
    pre1 = (Ab + jnp.dot(k, wk_ref[...], preferred_element_type=f32)
            + jnp.dot(q * k, wqk_ref[...], preferred_element_type=f32)
            + b1_ref[...])
    h1 = _prelu(pre1, a1_ref[...])
    h2 = _prelu(jnp.dot(h1, w2_ref[...], preferred_element_type=f32)
                + b2_ref[...], a2_ref[...])
    logits = jnp.dot(h2, wo_ref[...], preferred_element_type=f32) + bo_ref[...]
    ids = ids_ref[...]
    logits = jnp.where(ids == 0, f32(-2.0 ** 32 + 1), logits)
    l3 = logits.reshape(_TB, L, 1)
    mx = jnp.max(l3, axis=1, keepdims=True)
    E = jnp.exp(l3 - mx)
    numer = jnp.sum(E * k.reshape(_TB, L, D), axis=1)
    denom = jnp.sum(E, axis=1)
    user = numer / denom
    z = (jnp.dot(user, fwu_ref[...], preferred_element_type=f32)
         + jnp.dot(it, fwi_ref[...], preferred_element_type=f32)
         + jnp.dot(dn_ref[...], fwd_ref[...], preferred_element_type=f32)
         + jnp.dot(sp_ref[...], fws_ref[...], preferred_element_type=f32)
         + fb1_ref[...])
    g1 = _prelu(z, fa1_ref[...])
    g2 = _prelu(jnp.dot(g1, fw2_ref[...], preferred_element_type=f32)
                + fb2_ref[...], fa2_ref[...])
    y = jnp.dot(g2, fwo_ref[...], preferred_element_type=f32) + fbo_ref[...]
    out_ref[...] = jax.nn.sigmoid(y)


def _full(shape):
    return pl.BlockSpec(shape, lambda i: (0,) * len(shape))


def kernel(dense_inputs, sparse_inputs, seq_inputs, item_inputs, embed_sparse,
           embed_seq, att_W1, att_b1, att_a1, att_W2, att_b2, att_a2,
           att_Wout, att_bout, bn_gamma, bn_beta, ffn_W1, ffn_b1, ffn_a1,
           ffn_W2, ffn_b2, ffn_a2, Wout, bout):
    f32 = jnp.float32
    ids_flat = seq_inputs.reshape(B * L, 1).astype(jnp.int32)
    seq_idx = ids_flat.reshape(_SEQ_ROWS, _G)
    item_idx = item_inputs.astype(jnp.int32).reshape(_NW, 1, _G)
    sp_idx = (sparse_inputs.astype(jnp.int32)
              + (jnp.arange(NS, dtype=jnp.int32) * V)[None, :]
              ).reshape(_NW, _SP_PW, _G)
    sp_tbl = embed_sparse.reshape(NS * V, D)

    seq_g3, item_g3, sp_g3 = _build_sc_gather()(embed_seq, sp_tbl,
                                                seq_idx, item_idx, sp_idx)
    seq_g = seq_g3.reshape(B * L, D)
    item_g = item_g3.reshape(B, D)
    sp_g = sp_g3.reshape(B, NS * D)

    w1a, w1b, w1c, w1d = (att_W1[0:D], att_W1[D:2 * D],
                          att_W1[2 * D:3 * D], att_W1[3 * D:4 * D])
    wq = w1a + w1c
    wk = w1b - w1c
    wqk = w1d
    wbn = ffn_W1 * bn_gamma[:, None]
    fwu = wbn[0:D]
    fwi = wbn[D:2 * D]
    fwd = wbn[2 * D:2 * D + ND]
    fws = wbn[2 * D + ND:]
    fb1 = (ffn_b1 + bn_beta @ ffn_W1).reshape(1, 80)

    row = lambda v: v.reshape(1, -1).astype(f32)
    grid = B // _TB
    out = pl.pallas_call(
        _tc_body,
        grid=(grid,),
        in_specs=[
            pl.BlockSpec((_M, D), lambda i: (i, 0)),
            pl.BlockSpec((_M, 1), lambda i: (i, 0)),
            pl.BlockSpec((_TB, D), lambda i: (i, 0)),
            pl.BlockSpec((_TB, ND), lambda i: (i, 0)),
            pl.BlockSpec((_TB, NS * D), lambda i: (i, 0)),
            _full((D, 80)), _full((D, 80)), _full((D, 80)),
            _full((1, 80)), _full((1, 80)),
            _full((80, 40)), _full((1, 40)), _full((1, 40)),
            _full((40, 1)), _full((1, 1)),
            _full((D, 80)), _full((D, 80)), _full((ND, 80)),
            _full((NS * D, 80)), _full((1, 80)), _full((1, 80)),
            _full((80, 40)), _full((1, 40)), _full((1, 40)),
            _full((40, 1)), _full((1, 1)),
        ],
        out_specs=pl.BlockSpec((_TB, 1), lambda i: (i, 0)),
        out_shape=jax.ShapeDtypeStruct((B, 1), f32),
    )(seq_g, ids_flat, item_g, dense_inputs, sp_g,
      wq, wk, wqk, row(att_b1), row(att_a1),
      att_W2, row(att_b2), row(att_a2), att_Wout, row(att_bout),
      fwu, fwi, fwd, fws, fb1, row(ffn_a1),
      ffn_W2, row(ffn_b2), row(ffn_a2), Wout, row(bout))
    return out


# speedup vs baseline: 2.1658x; 1.2375x over previous
import functools

import jax
import jax.numpy as jnp
from jax import lax
from jax.experimental import pallas as pl
from jax.experimental.pallas import tpu as pltpu
from jax.experimental.pallas import tpu_sc as plsc

B = 4096
L = 200
V = 100000
D = 16
NS = 25
ND = 13

_NC = 2
_NSC = 16
_NW = _NC * _NSC

_G = 128
_SEQ_ROWS = B * L // _G
_SEQ_PW = _SEQ_ROWS // _NW
_K = 8
_SEQ_STEPS = _SEQ_PW // _K
_SP_ROWS = B * NS // _G
_SP_PW = _SP_ROWS // _NW

_NEG = -2.0 ** 32 + 1


def _sc_gather_body(seq_tbl, sp_tbl, seq_idx, item_idx, sp_idx,
                    seq_out, item_out, sp_out,
                    idx_s, rows_s, idx_p, rows_p, idx_i, rows_i, sem):
    wid = lax.axis_index("s") * _NC + lax.axis_index("c")

    seq_base = wid * _SEQ_PW

    def seq_step(i, carry):
        base = seq_base + i * _K
        pltpu.sync_copy(seq_idx.at[pl.ds(base, _K)], idx_s)
        cps = [pltpu.async_copy(seq_tbl.at[idx_s.at[j]],
                                rows_s.at[pl.ds(j * _G, _G)], sem)
               for j in range(_K)]
        for c in cps:
            c.wait()
        pltpu.sync_copy(rows_s, seq_out.at[pl.ds(base * _G, _K * _G)])
        return carry

    lax.fori_loop(0, _SEQ_STEPS, seq_step, 0)

    pltpu.sync_copy(sp_idx.at[wid], idx_p)
    cps = [pltpu.async_copy(sp_tbl.at[idx_p.at[j]],
                            rows_p.at[pl.ds(j * _G, _G)], sem)
           for j in range(13)]
    for c in cps:
        c.wait()
    cps = [pltpu.async_copy(sp_tbl.at[idx_p.at[j]],
                            rows_p.at[pl.ds(j * _G, _G)], sem)
           for j in range(13, _SP_PW)]
    for c in cps:
        c.wait()
    pltpu.sync_copy(rows_p, sp_out.at[pl.ds(wid * _SP_PW * _G, _SP_PW * _G)])

    pltpu.sync_copy(item_idx.at[wid], idx_i)
    pltpu.async_copy(seq_tbl.at[idx_i.at[0]], rows_i, sem).wait()
    pltpu.sync_copy(rows_i, item_out.at[pl.ds(wid * _G, _G)])


@functools.cache
def _build_sc_gather():
    return pl.kernel(
        _sc_gather_body,
        out_type=(
            jax.ShapeDtypeStruct((B * L, D), jnp.float32),
            jax.ShapeDtypeStruct((B, D), jnp.float32),
            jax.ShapeDtypeStruct((B * NS, D), jnp.float32),
        ),
        mesh=plsc.VectorSubcoreMesh(core_axis_name="c", subcore_axis_name="s",
                                    num_cores=_NC, num_subcores=_NSC),
        compiler_params=pltpu.CompilerParams(use_tc_tiling_on_sc=False),
        scratch_types=[
            pltpu.VMEM((_K, _G), jnp.int32),
            pltpu.VMEM((_K * _G, D), jnp.float32),
            pltpu.VMEM((_SP_PW, _G), jnp.int32),
            pltpu.VMEM((_SP_PW * _G, D), jnp.float32),
            pltpu.VMEM((1, _G), jnp.int32),
            pltpu.VMEM((_G, D), jnp.float32),
            pltpu.SemaphoreType.DMA,
        ],
    )


_TB = 64
_M = _TB * L
_MP = _M // 8
_RPB = L // 8


def _prelu(x, a):
    return jnp.where(x > 0, x, a * x)


def _tc_body(seq_ref, ids_ref, it_ref, dn_ref, sp_ref,
             wq_ref, t16_ref, t80_ref, wk8_ref, wqk8_ref, b1_ref, a1_ref,
             w28_ref, b2_ref, a2_ref, wo8_ref, bo_ref,
             rep_ref, fold_ref, coll_ref, ones8_ref,
             fwu_ref, fwi_ref, fwd_ref, fws_ref, fb1_ref, fa1_ref,
             fw2_ref, fb2_ref, fa2_ref, fwo_ref, fbo_ref,
             out_ref):
    f32 = jnp.float32
    dot = functools.partial(jnp.dot, preferred_element_type=f32)
    it = it_ref[...]
    k8 = seq_ref[...]
    ri = lax.broadcasted_iota(jnp.int32, (_MP, _G), 0)
    li = lax.broadcasted_iota(jnp.int32, (_MP, _G), 1)
    r25 = jnp.where(ri // _RPB == li, f32(1.0), f32(0.0))[:, :_TB]
    r16 = jnp.where(ri // 16 == li, f32(1.0), f32(0.0))[:, :_M // _G]
    A = dot(it, wq_ref[...])
    A8 = dot(r25, dot(A, t80_ref[...]))
    q8 = dot(r25, dot(it, t16_ref[...]))
    pre1 = (A8 + dot(k8, wk8_ref[...]) + dot(q8 * k8, wqk8_ref[...])
            + b1_ref[...])
    h1 = _prelu(pre1, a1_ref[...])
    h2 = _prelu(dot(h1, w28_ref[...]) + b2_ref[...], a2_ref[...])
    lg = dot(h2, wo8_ref[...]) + bo_ref[...]
    ids2 = ids_ref[...].reshape(_M // _G, _G)
    mf = jnp.where(ids2 == 0, f32(_NEG), f32(0.0))
    mf = dot(r16, mf)
    sel = (li // 8) == (ri % 16)
    lg = lg + dot(jnp.where(sel, mf, f32(0.0)), coll_ref[...])
    mx25 = jnp.max(lg.reshape(_TB, _RPB, 8), axis=1)
    for s in (4, 2, 1):
        mx25 = jnp.maximum(mx25, pltpu.roll(mx25, s, 1))
    mxr = dot(r25, mx25)
    E8 = jnp.exp(lg - mxr)
    d8 = dot(E8, ones8_ref[...])
    denom = jnp.sum(d8.reshape(_TB, _RPB, 1), axis=1)
    E128 = dot(E8, rep_ref[...])
    P = (E128 * k8).reshape(_TB, _RPB, _G)
    numer = dot(jnp.sum(P, axis=1), fold_ref[...])
    user = numer / denom
    z = (dot(user, fwu_ref[...]) + dot(it, fwi_ref[...])
         + dot(dn_ref[...], fwd_ref[...]) + dot(sp_ref[...], fws_ref[...])
         + fb1_ref[...])
    g1 = _prelu(z, fa1_ref[...])
    g2 = _prelu(dot(g1, fw2_ref[...]) + fb2_ref[...], fa2_ref[...])
    y = dot(g2, fwo_ref[...]) + fbo_ref[...]
    out_ref[...] = jax.nn.sigmoid(y)


def _full(shape):
    return pl.BlockSpec(shape, lambda i: (0,) * len(shape))


def kernel(dense_inputs, sparse_inputs, seq_inputs, item_inputs, embed_sparse,
           embed_seq, att_W1, att_b1, att_a1, att_W2, att_b2, att_a2,
           att_Wout, att_bout, bn_gamma, bn_beta, ffn_W1, ffn_b1, ffn_a1,
           ffn_W2, ffn_b2, ffn_a2, Wout, bout):
    f32 = jnp.float32
    seq_idx = seq_inputs.astype(jnp.int32).reshape(_SEQ_ROWS, _G)
    item_idx = item_inputs.astype(jnp.int32).reshape(_NW, 1, _G)
    sp_idx = (sparse_inputs.astype(jnp.int32)
              + (jnp.arange(NS, dtype=jnp.int32) * V)[None, :]
              ).reshape(_NW, _SP_PW, _G)
    sp_tbl = embed_sparse.reshape(NS * V, D)

    seq_g2, item_g, sp_g2 = _build_sc_gather()(embed_seq, sp_tbl,
                                               seq_idx, item_idx, sp_idx)
    grid = B // _TB
    seq_g = seq_g2.reshape(B * L * D // _G, _G)
    ids3 = seq_idx.reshape(grid, _M // _G, _G)
    sp_g = sp_g2.reshape(B, NS * D)

    w1a, w1b, w1c, w1d = (att_W1[0:D], att_W1[D:2 * D],
                          att_W1[2 * D:3 * D], att_W1[3 * D:4 * D])
    wq = w1a + w1c
    eye8 = jnp.eye(8, dtype=f32)
    eye16 = jnp.eye(16, dtype=f32)
    wk8 = jnp.kron(eye8, w1b - w1c)
    wqk8 = jnp.kron(eye8, w1d)
    w28 = jnp.kron(eye8, att_W2)
    wo8 = jnp.kron(eye8, att_Wout)
    t16 = jnp.tile(eye16, (1, 8))
    t80 = jnp.tile(jnp.eye(80, dtype=f32), (1, 8))
    rep = jnp.repeat(eye8, 16, axis=1)
    fold = jnp.tile(eye16, (8, 1))
    coll = jnp.tile(eye8, (16, 1))
    tile8 = lambda v: jnp.tile(v, 8).reshape(1, -1).astype(f32)

    wbn = ffn_W1 * bn_gamma[:, None]
    fwu = wbn[0:D]
    fwi = wbn[D:2 * D]
    fwd = wbn[2 * D:2 * D + ND]
    fws = wbn[2 * D + ND:]
    fb1 = (ffn_b1 + bn_beta @ ffn_W1).reshape(1, 80)

    row = lambda v: v.reshape(1, -1).astype(f32)
    out = pl.pallas_call(
        _tc_body,
        grid=(grid,),
        in_specs=[
            pl.BlockSpec((_M * D // _G, _G), lambda i: (i, 0)),
            pl.BlockSpec((1, _M // _G, _G), lambda i: (i, 0, 0)),
            pl.BlockSpec((_TB, D), lambda i: (i, 0)),
            pl.BlockSpec((_TB, ND), lambda i: (i, 0)),
            pl.BlockSpec((_TB, NS * D), lambda i: (i, 0)),
            _full((D, 80)), _full((D, _G)), _full((80, 640)),
            _full((_G, 640)), _full((_G, 640)),
            _full((1, 640)), _full((1, 640)),
            _full((640, 320)), _full((1, 320)), _full((1, 320)),
            _full((320, 8)), _full((1, 8)),
            _full((8, _G)), _full((_G, D)), _full((_G, 8)), _full((8, 1)),
            _full((D, 80)), _full((D, 80)), _full((ND, 80)),
            _full((NS * D, 80)), _full((1, 80)), _full((1, 80)),
            _full((80, 40)), _full((1, 40)), _full((1, 40)),
            _full((40, 1)), _full((1, 1)),
        ],
        out_specs=pl.BlockSpec((_TB, 1), lambda i: (i, 0)),
        out_shape=jax.ShapeDtypeStruct((B, 1), f32),
    )(seq_g, ids3, item_g, dense_inputs, sp_g,
      wq, t16, t80, wk8, wqk8, tile8(att_b1), tile8(att_a1),
      w28, tile8(att_b2), tile8(att_a2), wo8, tile8(att_bout),
      rep, fold, coll, jnp.ones((8, 1), f32),
      fwu, fwi, fwd, fws, fb1, row(ffn_a1),
      ffn_W2, row(ffn_b2), row(ffn_a2), Wout, row(bout))
    return out


# speedup vs baseline: 4.5934x; 2.1209x over previous
import functools

import jax
import jax.numpy as jnp
from jax import lax
from jax.experimental import pallas as pl
from jax.experimental.pallas import tpu as pltpu
from jax.experimental.pallas import tpu_sc as plsc

B = 4096
L = 200
V = 100000
D = 16
NS = 25
ND = 13

_NC = 2
_NSC = 16
_NW = _NC * _NSC

_G = 128
_SEQ_ROWS = B * L // _G
_SEQ_PW = _SEQ_ROWS // _NW
_K = 8
_SEQ_STEPS = _SEQ_PW // _K
_BPW = B // _NW
_F = NS * D

_NEG = -2.0 ** 32 + 1


def _sc_seq_body(seq_tbl, seq_idx, item_idx, seq_out, item_out,
                 idx_s, rows_s, idx_i, rows_i, sem):
    wid = lax.axis_index("s") * _NC + lax.axis_index("c")
    seq_base = wid * _SEQ_PW

    def seq_step(i, carry):
        base = seq_base + i * _K
        pltpu.sync_copy(seq_idx.at[pl.ds(base, _K)], idx_s)
        cps = [pltpu.async_copy(seq_tbl.at[idx_s.at[j]],
                                rows_s.at[pl.ds(j * _G, _G)], sem)
               for j in range(_K)]
        for c in cps:
            c.wait()
        pltpu.sync_copy(rows_s, seq_out.at[pl.ds(base * _G, _K * _G)])
        return carry

    lax.fori_loop(0, _SEQ_STEPS, seq_step, 0)

    pltpu.sync_copy(item_idx.at[wid], idx_i)
    pltpu.async_copy(seq_tbl.at[idx_i.at[0]], rows_i, sem).wait()
    pltpu.sync_copy(rows_i, item_out.at[pl.ds(wid * _G, _G)])


def _sc_sp_body(tbl1d, eidx, sp_out, idx_v, vals_v, sem):
    wid = lax.axis_index("s") * _NC + lax.axis_index("c")
    base = wid * _BPW
    pltpu.sync_copy(eidx.at[pl.ds(base, _BPW)], idx_v)

    def step(i, carry):
        cps = [pltpu.async_copy(tbl1d.at[idx_v.at[i * 16 + u]],
                                vals_v.at[i * 16 + u], sem)
               for u in range(16)]
        for c in cps:
            c.wait()
        return carry

    lax.fori_loop(0, _BPW // 16, step, 0)
    pltpu.sync_copy(vals_v, sp_out.at[pl.ds(base, _BPW)])


@functools.cache
def _build_sc_seq():
    return pl.kernel(
        _sc_seq_body,
        out_type=(
            jax.ShapeDtypeStruct((B * L, D), jnp.float32),
            jax.ShapeDtypeStruct((B, D), jnp.float32),
        ),
        mesh=plsc.VectorSubcoreMesh(core_axis_name="c", subcore_axis_name="s",
                                    num_cores=_NC, num_subcores=_NSC),
        compiler_params=pltpu.CompilerParams(use_tc_tiling_on_sc=False),
        scratch_types=[
            pltpu.VMEM((_K, _G), jnp.int32),
            pltpu.VMEM((_K * _G, D), jnp.float32),
            pltpu.VMEM((1, _G), jnp.int32),
            pltpu.VMEM((_G, D), jnp.float32),
            pltpu.SemaphoreType.DMA,
        ],
    )


@functools.cache
def _build_sc_sp():
    return pl.kernel(
        _sc_sp_body,
        out_type=jax.ShapeDtypeStruct((B, _F), jnp.float32),
        mesh=plsc.VectorSubcoreMesh(core_axis_name="c", subcore_axis_name="s",
                                    num_cores=_NC, num_subcores=_NSC),
        compiler_params=pltpu.CompilerParams(use_tc_tiling_on_sc=False),
        scratch_types=[
            pltpu.VMEM((_BPW, _F), jnp.int32),
            pltpu.VMEM((_BPW, _F), jnp.float32),
            pltpu.SemaphoreType.DMA,
        ],
    )


_TB = 128
_M = _TB * L
_MP = _M // 8
_RPB = L // 8
_IR = _M // _G


def _prelu(x, a):
    return jnp.where(x > 0, x, a * x)


def _tc_att_body(seq_ref, ids_ref, it_ref,
                 wq_ref, b1_ref, t16_ref, t80_ref, wk8_ref, wqk8_ref, a1_ref,
                 w28_ref, b2_ref, a2_ref, wo8_ref, bo_ref,
                 r25_ref, r16_ref, r25b_ref, sel_ref, coll_ref, ones8_ref,
                 rep_ref, fold_ref,
                 user_ref):
    f32 = jnp.float32
    dot = functools.partial(jnp.dot, preferred_element_type=f32)
    it = it_ref[...]
    k8 = seq_ref[...]
    A = dot(it, wq_ref[...]) + b1_ref[...]
    A8 = dot(r25_ref[...], dot(A, t80_ref[...]))
    q8 = dot(r25_ref[...], dot(it, t16_ref[...]))
    pre1 = A8 + dot(k8, wk8_ref[...]) + dot(q8 * k8, wqk8_ref[...])
    h1 = _prelu(pre1, a1_ref[...])
    h2 = _prelu(dot(h1, w28_ref[...]) + b2_ref[...], a2_ref[...])
    lg = dot(h2, wo8_ref[...]) + bo_ref[...]
    mf = jnp.where(ids_ref[...] == 0, f32(_NEG), f32(0.0))
    mf = dot(r16_ref[...], mf)
    lg = lg + dot(sel_ref[...] * mf, coll_ref[...])
    mx25 = jnp.max(lg.reshape(_TB, _RPB, 8), axis=1)
    for s in (4, 2, 1):
        mx25 = jnp.maximum(mx25, pltpu.roll(mx25, s, 1))
    E8 = jnp.exp(lg - dot(r25_ref[...], mx25))
    denom = dot(r25b_ref[...], dot(E8, ones8_ref[...]))
    E128 = dot(E8, rep_ref[...])
    numer = dot(dot(r25b_ref[...], E128 * k8), fold_ref[...])
    user_ref[...] = numer / denom


def _tc_ffn_body(user_ref, it_ref, dn_ref, sp_ref,
                 fwu_ref, fwi_ref, fwd_ref, fws_ref, fb1_ref, fa1_ref,
                 fw2_ref, fb2_ref, fa2_ref, fwo_ref, fbo_ref,
                 out_ref):
    f32 = jnp.float32
    dot = functools.partial(jnp.dot, preferred_element_type=f32)
    z = (dot(user_ref[...], fwu_ref[...]) + dot(it_ref[...], fwi_ref[...])
         + dot(dn_ref[...], fwd_ref[...]) + dot(sp_ref[...], fws_ref[...])
         + fb1_ref[...])
    g1 = _prelu(z, fa1_ref[...])
    g2 = _prelu(dot(g1, fw2_ref[...]) + fb2_ref[...], fa2_ref[...])
    y = dot(g2, fwo_ref[...]) + fbo_ref[...]
    out_ref[...] = jax.nn.sigmoid(y)


def _full(shape):
    return pl.BlockSpec(shape, lambda i: (0,) * len(shape))


def kernel(dense_inputs, sparse_inputs, seq_inputs, item_inputs, embed_sparse,
           embed_seq, att_W1, att_b1, att_a1, att_W2, att_b2, att_a2,
           att_Wout, att_bout, bn_gamma, bn_beta, ffn_W1, ffn_b1, ffn_a1,
           ffn_W2, ffn_b2, ffn_a2, Wout, bout):
    f32 = jnp.float32
    i32 = jnp.int32
    seq_idx = seq_inputs.astype(i32).reshape(_SEQ_ROWS, _G)
    item_idx = item_inputs.astype(i32).reshape(_NW, 1, _G)

    spT1d = embed_sparse.transpose(0, 2, 1).reshape(NS * D * V)
    off = (jnp.arange(NS, dtype=i32)[:, None] * (D * V)
           + jnp.arange(D, dtype=i32)[None, :] * V)
    eidx = (sparse_inputs.astype(i32)[:, :, None]
            + off[None, :, :]).reshape(B, _F)

    seq_g2, item_g = _build_sc_seq()(embed_seq, seq_idx, item_idx)
    sp_g = _build_sc_sp()(spT1d, eidx)

    grid = B // _TB
    seq_g = seq_g2.reshape(B * L * D // _G, _G)

    w1a, w1b, w1c, w1d = (att_W1[0:D], att_W1[D:2 * D],
                          att_W1[2 * D:3 * D], att_W1[3 * D:4 * D])
    wq = w1a + w1c
    eye8 = jnp.eye(8, dtype=f32)
    eye16 = jnp.eye(16, dtype=f32)
    wk8 = jnp.kron(eye8, w1b - w1c)
    wqk8 = jnp.kron(eye8, w1d)
    w28 = jnp.kron(eye8, att_W2)
    wo8 = jnp.kron(eye8, att_Wout)
    t16 = jnp.tile(eye16, (1, 8))
    t80 = jnp.tile(jnp.eye(80, dtype=f32), (1, 8))
    rep = jnp.repeat(eye8, 16, axis=1)
    fold = jnp.tile(eye16, (8, 1))
    coll = jnp.tile(eye8, (16, 1))
    tile8 = lambda v: jnp.tile(v, 8).reshape(1, -1).astype(f32)

    rr = jnp.arange(_MP, dtype=i32)
    r25 = (rr[:, None] // _RPB == jnp.arange(_TB, dtype=i32)[None, :]
           ).astype(f32)
    r16 = (rr[:, None] // 16 == jnp.arange(_IR, dtype=i32)[None, :]
           ).astype(f32)
    r25b = r25.T
    sel = (jnp.arange(_G, dtype=i32)[None, :] // 8 == rr[:, None] % 16
           ).astype(f32)

    wbn = ffn_W1 * bn_gamma[:, None]
    fwu = wbn[0:D]
    fwi = wbn[D:2 * D]
    fwd = wbn[2 * D:2 * D + ND]
    fws = wbn[2 * D + ND:]
    fb1 = (ffn_b1 + bn_beta @ ffn_W1).reshape(1, 80)
    row = lambda v: v.reshape(1, -1).astype(f32)

    user = pl.pallas_call(
        _tc_att_body,
        grid=(grid,),
        in_specs=[
            pl.BlockSpec((_M * D // _G, _G), lambda i: (i, 0)),
            pl.BlockSpec((_IR, _G), lambda i: (i, 0)),
            pl.BlockSpec((_TB, D), lambda i: (i, 0)),
            _full((D, 80)), _full((1, 80)), _full((D, _G)), _full((80, 640)),
            _full((_G, 640)), _full((_G, 640)), _full((1, 640)),
            _full((640, 320)), _full((1, 320)), _full((1, 320)),
            _full((320, 8)), _full((1, 8)),
            _full((_MP, _TB)), _full((_MP, _IR)), _full((_TB, _MP)),
            _full((_MP, _G)), _full((_G, 8)), _full((8, 1)),
            _full((8, _G)), _full((_G, D)),
        ],
        out_specs=pl.BlockSpec((_TB, D), lambda i: (i, 0)),
        out_shape=jax.ShapeDtypeStruct((B, D), f32),
    )(seq_g, seq_idx, item_g,
      wq, row(att_b1), t16, t80, wk8, wqk8, tile8(att_a1),
      w28, tile8(att_b2), tile8(att_a2), wo8, tile8(att_bout),
      r25, r16, r25b, sel, coll, jnp.ones((8, 1), f32),
      rep, fold)

    out = pl.pallas_call(
        _tc_ffn_body,
        grid=(grid,),
        in_specs=[
            pl.BlockSpec((_TB, D), lambda i: (i, 0)),
            pl.BlockSpec((_TB, D), lambda i: (i, 0)),
            pl.BlockSpec((_TB, ND), lambda i: (i, 0)),
            pl.BlockSpec((_TB, _F), lambda i: (i, 0)),
            _full((D, 80)), _full((D, 80)), _full((ND, 80)),
            _full((_F, 80)), _full((1, 80)), _full((1, 80)),
            _full((80, 40)), _full((1, 40)), _full((1, 40)),
            _full((40, 1)), _full((1, 1)),
        ],
        out_specs=pl.BlockSpec((_TB, 1), lambda i: (i, 0)),
        out_shape=jax.ShapeDtypeStruct((B, 1), f32),
    )(user, item_g, dense_inputs, sp_g,
      fwu, fwi, fwd, fws, fb1, row(ffn_a1),
      ffn_W2, row(ffn_b2), row(ffn_a2), Wout, row(bout))
    return out
